# Initial kernel scaffold; baseline (speedup 1.0000x reference)
import jax
import jax.numpy as jnp
from jax.experimental import pallas as pl


def kernel(idx0, idx1, idx2, emb0, emb1, emb2):
    raise NotImplementedError("write your pallas kernel here")


# speedup vs baseline: 2.5575x; 2.5575x over previous
import functools

import jax
import jax.numpy as jnp
from jax import lax
from jax.experimental import pallas as pl
from jax.experimental.pallas import tpu as pltpu
from jax.experimental.pallas import tpu_sc as plsc

B, L = 4096, 50
N = B * L
D = 32
OUTD = 3 * D
NW = 32
NPW = N // NW
C = 800
NCHUNK = NPW // C


def _make_kernel():
    info = plsc.get_sparse_core_info()
    nc = info.num_cores
    mesh = plsc.VectorSubcoreMesh(core_axis_name="c", subcore_axis_name="s")

    @functools.partial(
        pl.kernel,
        mesh=mesh,
        out_type=jax.ShapeDtypeStruct((N, OUTD), jnp.float32),
        compiler_params=pltpu.CompilerParams(use_tc_tiling_on_sc=False),
        scratch_types=[
            pltpu.VMEM((C,), jnp.int32),
            pltpu.VMEM((C,), jnp.int32),
            pltpu.VMEM((C,), jnp.int32),
            pltpu.VMEM((C, D), jnp.float32),
            pltpu.VMEM((C, D), jnp.float32),
            pltpu.VMEM((C, D), jnp.float32),
            pltpu.SemaphoreType.DMA,
        ],
    )
    def k(idx0_h, idx1_h, idx2_h, emb0_h, emb1_h, emb2_h, out_h,
          iv0, iv1, iv2, r0, r1, r2, sem):
        wid = lax.axis_index("s") * nc + lax.axis_index("c")
        base = wid * NPW

        def body(c, carry):
            cb = base + c * C
            pltpu.sync_copy(idx0_h.at[pl.ds(cb, C)], iv0)
            pltpu.sync_copy(idx1_h.at[pl.ds(cb, C)], iv1)
            pltpu.sync_copy(idx2_h.at[pl.ds(cb, C)], iv2)
            h0 = pltpu.async_copy(emb0_h.at[iv0], r0, sem)
            h1 = pltpu.async_copy(emb1_h.at[iv1], r1, sem)
            h2 = pltpu.async_copy(emb2_h.at[iv2], r2, sem)
            h0.wait()
            h1.wait()
            h2.wait()
            pltpu.sync_copy(r0, out_h.at[pl.ds(cb, C), pl.ds(0, D)])
            pltpu.sync_copy(r1, out_h.at[pl.ds(cb, C), pl.ds(D, D)])
            pltpu.sync_copy(r2, out_h.at[pl.ds(cb, C), pl.ds(2 * D, D)])
            return carry

        lax.fori_loop(0, NCHUNK, body, 0)

    return k


_kern = _make_kernel()


def kernel(idx0, idx1, idx2, emb0, emb1, emb2):
    out = _kern(idx0.reshape(N), idx1.reshape(N), idx2.reshape(N),
                emb0, emb1, emb2)
    return out.reshape(B, L, OUTD)


# speedup vs baseline: 2.5908x; 1.0130x over previous
import functools

import jax
import jax.numpy as jnp
from jax import lax
from jax.experimental import pallas as pl
from jax.experimental.pallas import tpu as pltpu
from jax.experimental.pallas import tpu_sc as plsc

B, L = 4096, 50
N = B * L
D = 32
OUTD = 3 * D
NW = 32
NPW = N // NW
C = 400
NCHUNK = NPW // C


def _make_kernel():
    info = plsc.get_sparse_core_info()
    nc = info.num_cores
    mesh = plsc.VectorSubcoreMesh(core_axis_name="c", subcore_axis_name="s")

    @functools.partial(
        pl.kernel,
        mesh=mesh,
        out_type=jax.ShapeDtypeStruct((N, OUTD), jnp.float32),
        compiler_params=pltpu.CompilerParams(use_tc_tiling_on_sc=False),
        scratch_types=[
            pltpu.VMEM((NPW,), jnp.int32),
            pltpu.VMEM((NPW,), jnp.int32),
            pltpu.VMEM((NPW,), jnp.int32),
            pltpu.VMEM((C, D), jnp.float32),
            pltpu.VMEM((C, D), jnp.float32),
            pltpu.VMEM((C, D), jnp.float32),
            pltpu.VMEM((C, D), jnp.float32),
            pltpu.VMEM((C, D), jnp.float32),
            pltpu.VMEM((C, D), jnp.float32),
            pltpu.SemaphoreType.DMA,
            pltpu.SemaphoreType.DMA,
        ],
    )
    def k(idx0_h, idx1_h, idx2_h, emb0_h, emb1_h, emb2_h, out_h,
          iv0, iv1, iv2, r00, r01, r02, r10, r11, r12, gsem, wsem):
        wid = lax.axis_index("s") * nc + lax.axis_index("c")
        base = wid * NPW

        embs = (emb0_h, emb1_h, emb2_h)
        ivs = (iv0, iv1, iv2)
        bufs = ((r00, r01, r02), (r10, r11, r12))

        hs = [pltpu.async_copy(ih.at[pl.ds(base, NPW)], iv, gsem)
              for ih, iv in zip((idx0_h, idx1_h, idx2_h), ivs)]
        for h in hs:
            h.wait()

        def fire_gather(c, bset):
            off = c * C
            for f in range(3):
                pltpu.async_copy(embs[f].at[ivs[f].at[pl.ds(off, C)]],
                                 bufs[bset][f], gsem)

        def wait_gather(bset):
            for f in range(3):
                pltpu.make_async_copy(embs[f].at[ivs[f].at[pl.ds(0, C)]],
                                      bufs[bset][f], gsem).wait()

        def fire_write(c, bset):
            cb = base + c * C
            for f in range(3):
                pltpu.async_copy(bufs[bset][f],
                                 out_h.at[pl.ds(cb, C), pl.ds(f * D, D)],
                                 wsem)

        def wait_write(bset):
            for f in range(3):
                pltpu.make_async_copy(bufs[bset][f],
                                      out_h.at[pl.ds(base, C),
                                               pl.ds(f * D, D)],
                                      wsem).wait()

        fire_gather(0, 0)
        fire_gather(1, 1)
        wait_gather(0)
        fire_write(0, 0)

        def body(g, carry):
            for bpar in range(2):
                c = 1 + g * 2 + bpar
                s = (1 + bpar) % 2
                wait_write(1 - s)
                fire_gather(c + 1, 1 - s)
                wait_gather(s)
                fire_write(c, s)
            return carry

        lax.fori_loop(0, (NCHUNK - 2) // 2, body, 0)

        cl = NCHUNK - 1
        sl = cl % 2
        wait_gather(sl)
        fire_write(cl, sl)
        wait_write(0)
        wait_write(1)

    return k


_kern = _make_kernel()


def kernel(idx0, idx1, idx2, emb0, emb1, emb2):
    out = _kern(idx0.reshape(N), idx1.reshape(N), idx2.reshape(N),
                emb0, emb1, emb2)
    return out.reshape(B, L, OUTD)
